# baseline (speedup 1.0000x reference)
import functools

import jax
import jax.numpy as jnp
from jax import lax
from jax.experimental import pallas as pl
from jax.experimental.pallas import tpu as pltpu
from jax.experimental.pallas import tpu_sc as plsc

N = 10000
DEG = 32
E = 320000
D = 128
DE = 16
H = 128

N_PAD = 10240
EP = N_PAD * DEG
ZROW0 = E
NZROWS = EP - E

NC, NS = 2, 16
NW = NC * NS

_SC_MESH = dict(mesh=plsc.VectorSubcoreMesh(core_axis_name="c",
                                            subcore_axis_name="s"))


def _wid():
    return lax.axis_index("s") * NC + lax.axis_index("c")


_G_CH = 128
_G_IB = 1024


def _sc_gather(cv, bond, adj_flat_p, abaj_flat_p):
    rows_w = EP // NW
    n_blocks = rows_w // _G_IB
    nsub = _G_IB // _G_CH

    def body(cv_hbm, adj_hbm, cvg_hbm, idxa, cvb0, cvb1, sg0, sg1, sw0, sw1):
        base = _wid() * rows_w
        bufs = (cvb0, cvb1)
        gsem = (sg0, sg1)
        wsem = (sw0, sw1)

        def block(bi, _):
            off = base + bi * _G_IB
            roff = pl.multiple_of(off // _G_CH, 8)
            pltpu.sync_copy(adj_hbm.at[pl.ds(roff, nsub)], idxa)
            g = {0: pltpu.async_copy(cv_hbm.at[idxa.at[0]], bufs[0], gsem[0])}
            for j in range(nsub):
                b = j & 1
                nb_ = (j + 1) & 1
                if j + 1 < nsub:
                    g[j + 1] = pltpu.async_copy(cv_hbm.at[idxa.at[j + 1]],
                                                bufs[nb_], gsem[nb_])
                g[j].wait()
                pltpu.sync_copy(bufs[b],
                                cvg_hbm.at[pl.ds(off + j * _G_CH, _G_CH)])
            return 0

        lax.fori_loop(0, n_blocks, block, 0)

    f = pl.kernel(
        body,
        out_type=jax.ShapeDtypeStruct((EP, H), jnp.int32),
        scratch_types=[
            pltpu.VMEM((nsub, _G_CH), jnp.int32),
            pltpu.VMEM((_G_CH, H), jnp.int32),
            pltpu.VMEM((_G_CH, H), jnp.int32),
            pltpu.SemaphoreType.DMA,
            pltpu.SemaphoreType.DMA,
            pltpu.SemaphoreType.DMA,
            pltpu.SemaphoreType.DMA,
        ],
        **_SC_MESH,
    )
    return f(cv, adj_flat_p.reshape(EP // _G_CH, _G_CH))


def _sc_gather_bond(bond, abaj_flat_p):
    rows_w = EP // NW
    n_blocks = rows_w // _G_IB
    nsub = _G_IB // _G_CH

    def body(bond_hbm, abaj_hbm, bondg_hbm, idxb, bb0, bb1, sg0, sg1):
        base = _wid() * rows_w
        bufs = (bb0, bb1)
        gsem = (sg0, sg1)

        def block(bi, _):
            off = base + bi * _G_IB
            roff = pl.multiple_of(off // _G_CH, 8)
            pltpu.sync_copy(abaj_hbm.at[pl.ds(roff, nsub)], idxb)
            g = {0: pltpu.async_copy(bond_hbm.at[idxb.at[0]], bufs[0],
                                     gsem[0])}
            for j in range(nsub):
                b = j & 1
                nb_ = (j + 1) & 1
                if j + 1 < nsub:
                    g[j + 1] = pltpu.async_copy(bond_hbm.at[idxb.at[j + 1]],
                                                bufs[nb_], gsem[nb_])
                g[j].wait()
                pltpu.sync_copy(bufs[b],
                                bondg_hbm.at[pl.ds(off + j * _G_CH, _G_CH)])
            return 0

        lax.fori_loop(0, n_blocks, block, 0)

    f = pl.kernel(
        body,
        out_type=jax.ShapeDtypeStruct((EP, DE), jnp.float32),
        scratch_types=[
            pltpu.VMEM((nsub, _G_CH), jnp.int32),
            pltpu.VMEM((_G_CH, DE), jnp.float32),
            pltpu.VMEM((_G_CH, DE), jnp.float32),
            pltpu.SemaphoreType.DMA,
            pltpu.SemaphoreType.DMA,
        ],
        compiler_params=pltpu.CompilerParams(use_tc_tiling_on_sc=False),
        **_SC_MESH,
    )
    return f(bond, abaj_flat_p.reshape(EP // _G_CH, _G_CH))


def _transforms_body(atom_ref, w_ref, b_ref, cv_ref, bu_ref):
    x = atom_ref[...]
    y = jnp.dot(x, w_ref[...], preferred_element_type=jnp.float32) + b_ref[...]
    cbits = lax.bitcast_convert_type(y[:, :H], jnp.int32)
    vbits = lax.bitcast_convert_type(y[:, H:256], jnp.int32)
    cv_ref[...] = ((cbits + 0x8000) & jnp.int32(-65536)) | \
        lax.shift_right_logical(vbits + 0x8000, 16)
    bu_ref[...] = y[:, 256:]


def _transforms(atom_pad, w_all, b_all):
    BA = 512
    grid = (N_PAD // BA,)
    return pl.pallas_call(
        _transforms_body,
        grid=grid,
        in_specs=[
            pl.BlockSpec((BA, D), lambda i: (i, 0)),
            pl.BlockSpec((D, 512), lambda i: (0, 0)),
            pl.BlockSpec((1, 512), lambda i: (0, 0)),
        ],
        out_specs=[
            pl.BlockSpec((BA, H), lambda i: (i, 0)),
            pl.BlockSpec((BA, 256), lambda i: (i, 0)),
        ],
        out_shape=[
            jax.ShapeDtypeStruct((N_PAD, H), jnp.int32),
            jax.ShapeDtypeStruct((N_PAD, 256), jnp.float32),
        ],
    )(atom_pad, w_all, b_all)


_W_BLK = 8
_W_G = 8
_W_NR = NW // _W_G
_W_RANGE = EP // _W_NR
_W_SLC = _W_RANGE // _W_G


def _sc_winner(abaj_w):
    rows_m = (EP // 128) // _W_G
    n_blocks = rows_m // _W_BLK

    def body(abaj_hbm, stage_hbm, win, idxw, sem):
        c = lax.axis_index("c")
        s = lax.axis_index("s")
        g = s // _W_G
        m = s % _W_G
        lo = (c * 2 + g) * _W_RANGE
        iota = lax.iota(jnp.int32, 16)

        def init(i, _):
            for u in range(8):
                win[pl.ds((i * 8 + u) * 16, 16)] = jnp.full((16,), -1,
                                                            jnp.int32)
            return 0

        lax.fori_loop(0, _W_RANGE // 128, init, 0)

        row0 = m * rows_m

        def block(bi, _):
            roff = pl.multiple_of(row0 + bi * _W_BLK, 8)
            pltpu.sync_copy(abaj_hbm.at[pl.ds(roff, _W_BLK)], idxw)

            for r in range(_W_BLK):
                for cc in range(8):
                    k_base = (row0 + bi * _W_BLK + r) * 128 + cc * 16
                    e_vec = idxw[r, pl.ds(cc * 16, 16)]
                    k_vec = k_base + iota
                    inr = (e_vec >= lo) & (e_vec < lo + _W_RANGE)
                    rel = jnp.where(inr, e_vec - lo, 0)
                    plsc.store_scatter(win, [rel], k_vec, mask=inr)
                    stored = plsc.load_gather(win, [rel], mask=inr)
                    needi = (inr & (stored < k_vec)).astype(jnp.int32)

                    def fix_cond(cr):
                        return cr[0] > 0

                    def fix_body(cr):
                        _, ni = cr
                        mm = ni > 0
                        plsc.store_scatter(win, [rel], k_vec, mask=mm)
                        st = plsc.load_gather(win, [rel], mask=mm)
                        ni2 = (mm & (st < k_vec)).astype(jnp.int32)
                        return (jnp.sum(ni2), ni2)

                    lax.while_loop(fix_cond, fix_body,
                                   (jnp.sum(needi), needi))
            return 0

        lax.fori_loop(0, n_blocks, block, 0)

        pltpu.sync_copy(win, stage_hbm.at[pl.ds((c * NS + s) * _W_RANGE, _W_RANGE)])

    f = pl.kernel(
        body,
        out_type=jax.ShapeDtypeStruct((NW * _W_RANGE,), jnp.int32),
        scratch_types=[
            pltpu.VMEM((_W_RANGE,), jnp.int32),
            pltpu.VMEM((_W_BLK, 128), jnp.int32),
            pltpu.SemaphoreType.DMA,
        ],
        compiler_params=pltpu.CompilerParams(needs_layout_passes=False),
        **_SC_MESH,
    )
    stage = f(abaj_w)

    def mbody(stage_hbm, win_hbm, tmp, outb, sem):
        c = lax.axis_index("c")
        s = lax.axis_index("s")
        g = s // _W_G
        m = s % _W_G
        lo = (c * 2 + g) * _W_RANGE
        sl = m * _W_SLC
        out_lo = lo + sl
        iota = lax.iota(jnp.int32, 16)
        pltpu.sync_copy(
            stage_hbm.at[pl.ds((c * NS + g * _W_G) * _W_RANGE + sl, _W_SLC)],
            outb)
        for src in range(1, _W_G):
            pltpu.sync_copy(
                stage_hbm.at[pl.ds((c * NS + g * _W_G + src) * _W_RANGE + sl,
                                   _W_SLC)], tmp)

            def mx(i, _):
                for u in range(8):
                    o = (i * 8 + u) * 16
                    outb[pl.ds(o, 16)] = jnp.maximum(outb[pl.ds(o, 16)],
                                                     tmp[pl.ds(o, 16)])
                return 0

            lax.fori_loop(0, _W_SLC // 128, mx, 0)

        def fin(i, _):
            for u in range(8):
                o = (i * 8 + u) * 16
                v = outb[pl.ds(o, 16)]
                sent = E + lax.rem(out_lo + o + iota, NZROWS)
                outb[pl.ds(o, 16)] = jnp.where(v < 0, sent, v)
            return 0

        lax.fori_loop(0, _W_SLC // 128, fin, 0)
        pltpu.sync_copy(outb, win_hbm.at[pl.ds(out_lo, _W_SLC)])

    fm = pl.kernel(
        mbody,
        out_type=jax.ShapeDtypeStruct((EP,), jnp.int32),
        scratch_types=[
            pltpu.VMEM((_W_SLC,), jnp.int32),
            pltpu.VMEM((_W_SLC,), jnp.int32),
            pltpu.SemaphoreType.DMA,
        ],
        **_SC_MESH,
    )
    return fm(stage)


def _sc_bond_gather(winner, nb_flat):
    n_groups = E // 128

    def body(win_hbm, nb_hbm, out_hbm, idx0, idx1, db0, db1, sg0, sg1):
        wid = _wid()
        per = n_groups // NW
        extra = n_groups - per * NW
        base_g = wid * per
        idxs = (idx0, idx1)
        bufs = (db0, db1)
        gsem = (sg0, sg1)

        def pair(pi, _):
            off0 = pl.multiple_of((base_g + 2 * pi) * 128, 128)
            off1 = pl.multiple_of((base_g + 2 * pi + 1) * 128, 128)
            pltpu.sync_copy(win_hbm.at[pl.ds(off0, 128)], idxs[0])
            g0 = pltpu.async_copy(nb_hbm.at[idxs[0]], bufs[0], gsem[0])
            pltpu.sync_copy(win_hbm.at[pl.ds(off1, 128)], idxs[1])
            g1 = pltpu.async_copy(nb_hbm.at[idxs[1]], bufs[1], gsem[1])
            g0.wait()
            pltpu.sync_copy(bufs[0], out_hbm.at[pl.ds(off0, 128)])
            g1.wait()
            pltpu.sync_copy(bufs[1], out_hbm.at[pl.ds(off1, 128)])
            return 0

        lax.fori_loop(0, per // 2, pair, 0)

        @pl.when(wid < extra)
        def _tail():
            off = pl.multiple_of((NW * per + wid) * 128, 128)
            pltpu.sync_copy(win_hbm.at[pl.ds(off, 128)], idxs[0])
            pltpu.async_copy(nb_hbm.at[idxs[0]], bufs[0], gsem[0]).wait()
            pltpu.sync_copy(bufs[0], out_hbm.at[pl.ds(off, 128)])

    f = pl.kernel(
        body,
        out_type=jax.ShapeDtypeStruct((E, H), jnp.float32),
        scratch_types=[
            pltpu.VMEM((128,), jnp.int32),
            pltpu.VMEM((128,), jnp.int32),
            pltpu.VMEM((128, H), jnp.float32),
            pltpu.VMEM((128, H), jnp.float32),
            pltpu.SemaphoreType.DMA,
            pltpu.SemaphoreType.DMA,
        ],
        **_SC_MESH,
    )
    return f(winner, nb_flat)


_BA = 128


def _edge_body(cvg_ref, bondg_ref, bu_ref, awt_ref, ab_ref, nb_ref, ao_ref):
    i = pl.program_id(0)
    px = cvg_ref[...]
    cx = lax.bitcast_convert_type(px & jnp.int32(-65536), jnp.float32)
    vn = lax.bitcast_convert_type(lax.shift_left(px, 16), jnp.float32)
    bg = bondg_ref[...].reshape(_BA * DEG, DE)
    ae = jnp.dot(bg, awt_ref[...], preferred_element_type=jnp.float32)
    syn = ae.reshape(_BA, DEG, H) + ab_ref[...][None, :, :] + cx \
        + bu_ref[:, :H][:, None, :]
    ids = i * _BA + lax.broadcasted_iota(jnp.int32, (_BA, 1, 1), 0)
    nb_ref[...] = jnp.where(ids < N, jax.nn.relu(syn), 0.0)
    gates = jax.nn.sigmoid(syn)
    agg = jnp.sum(gates * vn, axis=1)
    ao_ref[...] = jax.nn.relu(bu_ref[:, H:] + agg)


def _tc_edge(cvg, bondg, bu, A_w, A_b):
    grid = (N_PAD // _BA,)
    nb, ao = pl.pallas_call(
        _edge_body,
        grid=grid,
        in_specs=[
            pl.BlockSpec((_BA, DEG, H), lambda i: (i, 0, 0)),
            pl.BlockSpec((_BA, DEG, DE), lambda i: (i, 0, 0)),
            pl.BlockSpec((_BA, 256), lambda i: (i, 0)),
            pl.BlockSpec((DE, H), lambda i: (0, 0)),
            pl.BlockSpec((1, H), lambda i: (0, 0)),
        ],
        out_specs=[
            pl.BlockSpec((_BA, DEG, H), lambda i: (i, 0, 0)),
            pl.BlockSpec((_BA, H), lambda i: (i, 0)),
        ],
        out_shape=[
            jax.ShapeDtypeStruct((N_PAD, DEG, H), jnp.float32),
            jax.ShapeDtypeStruct((N_PAD, H), jnp.float32),
        ],
    )(cvg.reshape(N_PAD, DEG, H), bondg.reshape(N_PAD, DEG, DE), bu,
      A_w.T, A_b[None, :])
    return nb, ao


def kernel(atom_feature_matrix, bond_feature_matrix, atom_adjacency_list,
           atom_bond_adjacency_list, U_w, U_b, V_w, V_b, A_w, A_b, B_w, B_b,
           C_w, C_b):
    atom_pad = jnp.concatenate(
        [atom_feature_matrix, jnp.zeros((N_PAD - N, D), jnp.float32)], axis=0)
    w_all = jnp.concatenate([C_w.T, V_w.T, B_w.T, U_w.T], axis=1)
    b_all = jnp.concatenate([C_b, V_b, B_b, U_b])[None, :]

    cv, bu = _transforms(atom_pad, w_all, b_all)

    adj_flat = atom_adjacency_list.reshape(-1)
    abaj_flat = atom_bond_adjacency_list.reshape(-1)

    pad_ids = (jnp.arange(N_PAD * DEG - E, dtype=jnp.int32) % N)
    adj_flat_p = jnp.concatenate([adj_flat, pad_ids])
    abaj_flat_p = jnp.concatenate([abaj_flat, pad_ids])

    cvg = _sc_gather(cv, bond_feature_matrix, adj_flat_p, abaj_flat_p)
    bondg = _sc_gather_bond(bond_feature_matrix, abaj_flat_p)

    nb, ao = _tc_edge(cvg, bondg, bu, A_w, A_b)
    atom_out = ao[:N]

    abaj_w = jnp.concatenate(
        [abaj_flat, jnp.full((EP - E,), E, jnp.int32)]).reshape(EP // 128, 128)
    winner = _sc_winner(abaj_w)
    bond_out = _sc_bond_gather(winner, nb.reshape(EP, H))
    return (atom_out, bond_out)


# speedup vs baseline: 1.0864x; 1.0864x over previous
import functools

import jax
import jax.numpy as jnp
from jax import lax
from jax.experimental import pallas as pl
from jax.experimental.pallas import tpu as pltpu
from jax.experimental.pallas import tpu_sc as plsc

N = 10000
DEG = 32
E = 320000
D = 128
DE = 16
H = 128

N_PAD = 10240
EP = N_PAD * DEG
ZROW0 = E
NZROWS = EP - E

NC, NS = 2, 16
NW = NC * NS

_SC_MESH = dict(mesh=plsc.VectorSubcoreMesh(core_axis_name="c",
                                            subcore_axis_name="s"))


def _wid():
    return lax.axis_index("s") * NC + lax.axis_index("c")


_G_CH = 128
_G_IB = 1024


def _sc_gather(cv, bond, adj_flat_p, abaj_flat_p):
    rows_w = EP // NW
    n_blocks = rows_w // _G_IB
    nsub = _G_IB // _G_CH

    def body(cv_hbm, adj_hbm, cvg_hbm, idxa, cvb0, cvb1, sg0, sg1, sw0, sw1):
        base = _wid() * rows_w
        bufs = (cvb0, cvb1)
        gsem = (sg0, sg1)
        wsem = (sw0, sw1)

        def block(bi, _):
            off = base + bi * _G_IB
            roff = pl.multiple_of(off // _G_CH, 8)
            pltpu.sync_copy(adj_hbm.at[pl.ds(roff, nsub)], idxa)
            g = {0: pltpu.async_copy(cv_hbm.at[idxa.at[0]], bufs[0], gsem[0])}
            for j in range(nsub):
                b = j & 1
                nb_ = (j + 1) & 1
                if j + 1 < nsub:
                    g[j + 1] = pltpu.async_copy(cv_hbm.at[idxa.at[j + 1]],
                                                bufs[nb_], gsem[nb_])
                g[j].wait()
                pltpu.sync_copy(bufs[b],
                                cvg_hbm.at[pl.ds(off + j * _G_CH, _G_CH)])
            return 0

        lax.fori_loop(0, n_blocks, block, 0)

    f = pl.kernel(
        body,
        out_type=jax.ShapeDtypeStruct((EP, H), jnp.int32),
        scratch_types=[
            pltpu.VMEM((nsub, _G_CH), jnp.int32),
            pltpu.VMEM((_G_CH, H), jnp.int32),
            pltpu.VMEM((_G_CH, H), jnp.int32),
            pltpu.SemaphoreType.DMA,
            pltpu.SemaphoreType.DMA,
            pltpu.SemaphoreType.DMA,
            pltpu.SemaphoreType.DMA,
        ],
        **_SC_MESH,
    )
    return f(cv, adj_flat_p.reshape(EP // _G_CH, _G_CH))


def _sc_gather_bond(bond, abaj_flat_p):
    rows_w = EP // NW
    n_blocks = rows_w // _G_IB
    nsub = _G_IB // _G_CH

    def body(bond_hbm, abaj_hbm, bondg_hbm, idxb, bb0, bb1, sg0, sg1):
        base = _wid() * rows_w
        bufs = (bb0, bb1)
        gsem = (sg0, sg1)

        def block(bi, _):
            off = base + bi * _G_IB
            roff = pl.multiple_of(off // _G_CH, 8)
            pltpu.sync_copy(abaj_hbm.at[pl.ds(roff, nsub)], idxb)
            g = {0: pltpu.async_copy(bond_hbm.at[idxb.at[0]], bufs[0],
                                     gsem[0])}
            for j in range(nsub):
                b = j & 1
                nb_ = (j + 1) & 1
                if j + 1 < nsub:
                    g[j + 1] = pltpu.async_copy(bond_hbm.at[idxb.at[j + 1]],
                                                bufs[nb_], gsem[nb_])
                g[j].wait()
                pltpu.sync_copy(bufs[b],
                                bondg_hbm.at[pl.ds(off + j * _G_CH, _G_CH)])
            return 0

        lax.fori_loop(0, n_blocks, block, 0)

    f = pl.kernel(
        body,
        out_type=jax.ShapeDtypeStruct((EP, DE), jnp.float32),
        scratch_types=[
            pltpu.VMEM((nsub, _G_CH), jnp.int32),
            pltpu.VMEM((_G_CH, DE), jnp.float32),
            pltpu.VMEM((_G_CH, DE), jnp.float32),
            pltpu.SemaphoreType.DMA,
            pltpu.SemaphoreType.DMA,
        ],
        compiler_params=pltpu.CompilerParams(use_tc_tiling_on_sc=False),
        **_SC_MESH,
    )
    return f(bond, abaj_flat_p.reshape(EP // _G_CH, _G_CH))


def _transforms_body(atom_ref, w_ref, b_ref, cv_ref, bu_ref):
    x = atom_ref[...]
    y = jnp.dot(x, w_ref[...], preferred_element_type=jnp.float32) + b_ref[...]
    cbits = lax.bitcast_convert_type(y[:, :H], jnp.int32)
    vbits = lax.bitcast_convert_type(y[:, H:256], jnp.int32)
    cv_ref[...] = ((cbits + 0x8000) & jnp.int32(-65536)) | \
        lax.shift_right_logical(vbits + 0x8000, 16)
    bu_ref[...] = y[:, 256:]


def _transforms(atom_pad, w_all, b_all):
    BA = 512
    grid = (N_PAD // BA,)
    return pl.pallas_call(
        _transforms_body,
        grid=grid,
        in_specs=[
            pl.BlockSpec((BA, D), lambda i: (i, 0)),
            pl.BlockSpec((D, 512), lambda i: (0, 0)),
            pl.BlockSpec((1, 512), lambda i: (0, 0)),
        ],
        out_specs=[
            pl.BlockSpec((BA, H), lambda i: (i, 0)),
            pl.BlockSpec((BA, 256), lambda i: (i, 0)),
        ],
        out_shape=[
            jax.ShapeDtypeStruct((N_PAD, H), jnp.int32),
            jax.ShapeDtypeStruct((N_PAD, 256), jnp.float32),
        ],
    )(atom_pad, w_all, b_all)


_W_BLK = 8
_W_G = 8
_W_NR = NW // _W_G
_W_RANGE = EP // _W_NR
_W_SLC = _W_RANGE // _W_G


def _sc_winner(abaj_w):
    rows_m = (EP // 128) // _W_G
    n_blocks = rows_m // _W_BLK

    def body(abaj_hbm, stage_hbm, win, idxw, sem):
        c = lax.axis_index("c")
        s = lax.axis_index("s")
        g = s // _W_G
        m = s % _W_G
        lo = (c * 2 + g) * _W_RANGE
        iota = lax.iota(jnp.int32, 16)

        def init(i, _):
            for u in range(8):
                win[pl.ds((i * 8 + u) * 16, 16)] = jnp.full((16,), -1,
                                                            jnp.int32)
            return 0

        lax.fori_loop(0, _W_RANGE // 128, init, 0)

        row0 = m * rows_m

        def block(bi, _):
            roff = pl.multiple_of(row0 + bi * _W_BLK, 8)
            pltpu.sync_copy(abaj_hbm.at[pl.ds(roff, _W_BLK)], idxw)

            def row(r, _):
                for cc in range(8):
                    k_base = (row0 + bi * _W_BLK + r) * 128 + cc * 16
                    e_vec = idxw[r, pl.ds(cc * 16, 16)]
                    k_vec = k_base + iota
                    inr = (e_vec >= lo) & (e_vec < lo + _W_RANGE)
                    rel = jnp.where(inr, e_vec - lo, 0)
                    plsc.store_scatter(win, [rel], k_vec, mask=inr)
                    stored = plsc.load_gather(win, [rel], mask=inr)
                    needi = (inr & (stored < k_vec)).astype(jnp.int32)

                    def fix_cond(cr):
                        return cr[0] > 0

                    def fix_body(cr):
                        _, ni = cr
                        mm = ni > 0
                        plsc.store_scatter(win, [rel], k_vec, mask=mm)
                        st = plsc.load_gather(win, [rel], mask=mm)
                        ni2 = (mm & (st < k_vec)).astype(jnp.int32)
                        return (jnp.sum(ni2), ni2)

                    lax.while_loop(fix_cond, fix_body,
                                   (jnp.sum(needi), needi))
                return 0

            lax.fori_loop(0, _W_BLK, row, 0)
            return 0

        lax.fori_loop(0, n_blocks, block, 0)

        pltpu.sync_copy(win, stage_hbm.at[pl.ds((c * NS + s) * _W_RANGE, _W_RANGE)])

    f = pl.kernel(
        body,
        out_type=jax.ShapeDtypeStruct((NW * _W_RANGE,), jnp.int32),
        scratch_types=[
            pltpu.VMEM((_W_RANGE,), jnp.int32),
            pltpu.VMEM((_W_BLK, 128), jnp.int32),
            pltpu.SemaphoreType.DMA,
        ],
        compiler_params=pltpu.CompilerParams(needs_layout_passes=False),
        **_SC_MESH,
    )
    stage = f(abaj_w)

    def mbody(stage_hbm, win_hbm, tmp, outb, sem):
        c = lax.axis_index("c")
        s = lax.axis_index("s")
        g = s // _W_G
        m = s % _W_G
        lo = (c * 2 + g) * _W_RANGE
        sl = m * _W_SLC
        out_lo = lo + sl
        iota = lax.iota(jnp.int32, 16)
        pltpu.sync_copy(
            stage_hbm.at[pl.ds((c * NS + g * _W_G) * _W_RANGE + sl, _W_SLC)],
            outb)
        for src in range(1, _W_G):
            pltpu.sync_copy(
                stage_hbm.at[pl.ds((c * NS + g * _W_G + src) * _W_RANGE + sl,
                                   _W_SLC)], tmp)

            def mx(i, _):
                for u in range(8):
                    o = (i * 8 + u) * 16
                    outb[pl.ds(o, 16)] = jnp.maximum(outb[pl.ds(o, 16)],
                                                     tmp[pl.ds(o, 16)])
                return 0

            lax.fori_loop(0, _W_SLC // 128, mx, 0)

        def fin(i, _):
            for u in range(8):
                o = (i * 8 + u) * 16
                v = outb[pl.ds(o, 16)]
                sent = E + lax.rem(out_lo + o + iota, NZROWS)
                outb[pl.ds(o, 16)] = jnp.where(v < 0, sent, v)
            return 0

        lax.fori_loop(0, _W_SLC // 128, fin, 0)
        pltpu.sync_copy(outb, win_hbm.at[pl.ds(out_lo, _W_SLC)])

    fm = pl.kernel(
        mbody,
        out_type=jax.ShapeDtypeStruct((EP,), jnp.int32),
        scratch_types=[
            pltpu.VMEM((_W_SLC,), jnp.int32),
            pltpu.VMEM((_W_SLC,), jnp.int32),
            pltpu.SemaphoreType.DMA,
        ],
        **_SC_MESH,
    )
    return fm(stage)


def _sc_bond_gather(winner, nb_flat):
    n_groups = E // 128

    def body(win_hbm, nb_hbm, out_hbm, idx0, idx1, db0, db1, sg0, sg1):
        wid = _wid()
        per = n_groups // NW
        extra = n_groups - per * NW
        base_g = wid * per
        idxs = (idx0, idx1)
        bufs = (db0, db1)
        gsem = (sg0, sg1)

        def pair(pi, _):
            off0 = pl.multiple_of((base_g + 2 * pi) * 128, 128)
            off1 = pl.multiple_of((base_g + 2 * pi + 1) * 128, 128)
            pltpu.sync_copy(win_hbm.at[pl.ds(off0, 128)], idxs[0])
            g0 = pltpu.async_copy(nb_hbm.at[idxs[0]], bufs[0], gsem[0])
            pltpu.sync_copy(win_hbm.at[pl.ds(off1, 128)], idxs[1])
            g1 = pltpu.async_copy(nb_hbm.at[idxs[1]], bufs[1], gsem[1])
            g0.wait()
            pltpu.sync_copy(bufs[0], out_hbm.at[pl.ds(off0, 128)])
            g1.wait()
            pltpu.sync_copy(bufs[1], out_hbm.at[pl.ds(off1, 128)])
            return 0

        lax.fori_loop(0, per // 2, pair, 0)

        @pl.when(wid < extra)
        def _tail():
            off = pl.multiple_of((NW * per + wid) * 128, 128)
            pltpu.sync_copy(win_hbm.at[pl.ds(off, 128)], idxs[0])
            pltpu.async_copy(nb_hbm.at[idxs[0]], bufs[0], gsem[0]).wait()
            pltpu.sync_copy(bufs[0], out_hbm.at[pl.ds(off, 128)])

    f = pl.kernel(
        body,
        out_type=jax.ShapeDtypeStruct((E, H), jnp.float32),
        scratch_types=[
            pltpu.VMEM((128,), jnp.int32),
            pltpu.VMEM((128,), jnp.int32),
            pltpu.VMEM((128, H), jnp.float32),
            pltpu.VMEM((128, H), jnp.float32),
            pltpu.SemaphoreType.DMA,
            pltpu.SemaphoreType.DMA,
        ],
        **_SC_MESH,
    )
    return f(winner, nb_flat)


_BA = 256


def _edge_body(cvg_ref, bondg_ref, bu_ref, awt_ref, ab_ref, nb_ref, ao_ref):
    i = pl.program_id(0)
    px = cvg_ref[...]
    cx = lax.bitcast_convert_type(px & jnp.int32(-65536), jnp.float32)
    vn = lax.bitcast_convert_type(lax.shift_left(px, 16), jnp.float32)
    bg = bondg_ref[...].reshape(_BA * DEG, DE)
    ae = jnp.dot(bg, awt_ref[...], preferred_element_type=jnp.float32)
    syn = ae.reshape(_BA, DEG, H) + ab_ref[...][None, :, :] + cx \
        + bu_ref[:, :H][:, None, :]
    ids = i * _BA + lax.broadcasted_iota(jnp.int32, (_BA, 1, 1), 0)
    nb_ref[...] = jnp.where(ids < N, jax.nn.relu(syn), 0.0)
    gates = jax.nn.sigmoid(syn)
    agg = jnp.sum(gates * vn, axis=1)
    ao_ref[...] = jax.nn.relu(bu_ref[:, H:] + agg)


def _tc_edge(cvg, bondg, bu, A_w, A_b):
    grid = (N_PAD // _BA,)
    nb, ao = pl.pallas_call(
        _edge_body,
        grid=grid,
        in_specs=[
            pl.BlockSpec((_BA, DEG, H), lambda i: (i, 0, 0)),
            pl.BlockSpec((_BA, DEG, DE), lambda i: (i, 0, 0)),
            pl.BlockSpec((_BA, 256), lambda i: (i, 0)),
            pl.BlockSpec((DE, H), lambda i: (0, 0)),
            pl.BlockSpec((1, H), lambda i: (0, 0)),
        ],
        out_specs=[
            pl.BlockSpec((_BA, DEG, H), lambda i: (i, 0, 0)),
            pl.BlockSpec((_BA, H), lambda i: (i, 0)),
        ],
        out_shape=[
            jax.ShapeDtypeStruct((N_PAD, DEG, H), jnp.float32),
            jax.ShapeDtypeStruct((N_PAD, H), jnp.float32),
        ],
    )(cvg.reshape(N_PAD, DEG, H), bondg.reshape(N_PAD, DEG, DE), bu,
      A_w.T, A_b[None, :])
    return nb, ao


def kernel(atom_feature_matrix, bond_feature_matrix, atom_adjacency_list,
           atom_bond_adjacency_list, U_w, U_b, V_w, V_b, A_w, A_b, B_w, B_b,
           C_w, C_b):
    atom_pad = jnp.concatenate(
        [atom_feature_matrix, jnp.zeros((N_PAD - N, D), jnp.float32)], axis=0)
    w_all = jnp.concatenate([C_w.T, V_w.T, B_w.T, U_w.T], axis=1)
    b_all = jnp.concatenate([C_b, V_b, B_b, U_b])[None, :]

    cv, bu = _transforms(atom_pad, w_all, b_all)

    adj_flat = atom_adjacency_list.reshape(-1)
    abaj_flat = atom_bond_adjacency_list.reshape(-1)

    pad_ids = (jnp.arange(N_PAD * DEG - E, dtype=jnp.int32) % N)
    adj_flat_p = jnp.concatenate([adj_flat, pad_ids])
    abaj_flat_p = jnp.concatenate([abaj_flat, pad_ids])

    cvg = _sc_gather(cv, bond_feature_matrix, adj_flat_p, abaj_flat_p)
    bondg = _sc_gather_bond(bond_feature_matrix, abaj_flat_p)

    nb, ao = _tc_edge(cvg, bondg, bu, A_w, A_b)
    atom_out = ao[:N]

    abaj_w = jnp.concatenate(
        [abaj_flat, jnp.full((EP - E,), E, jnp.int32)]).reshape(EP // 128, 128)
    winner = _sc_winner(abaj_w)
    bond_out = _sc_bond_gather(winner, nb.reshape(EP, H))
    return (atom_out, bond_out)


# speedup vs baseline: 1.0929x; 1.0060x over previous
import functools

import jax
import jax.numpy as jnp
from jax import lax
from jax.experimental import pallas as pl
from jax.experimental.pallas import tpu as pltpu
from jax.experimental.pallas import tpu_sc as plsc

N = 10000
DEG = 32
E = 320000
D = 128
DE = 16
H = 128

N_PAD = 10240
EP = N_PAD * DEG
ZROW0 = E
NZROWS = EP - E

NC, NS = 2, 16
NW = NC * NS

_SC_MESH = dict(mesh=plsc.VectorSubcoreMesh(core_axis_name="c",
                                            subcore_axis_name="s"))


def _wid():
    return lax.axis_index("s") * NC + lax.axis_index("c")


_G_CH = 128
_G_IB = 1024


def _sc_gather(cv, bond, adj_flat_p, abaj_flat_p):
    rows_w = EP // NW
    n_blocks = rows_w // _G_IB
    nsub = _G_IB // _G_CH

    def body(cv_hbm, adj_hbm, cvg_hbm, idxa, cvb0, cvb1, sg0, sg1, sw0, sw1):
        base = _wid() * rows_w
        bufs = (cvb0, cvb1)
        gsem = (sg0, sg1)
        wsem = (sw0, sw1)

        def block(bi, _):
            off = base + bi * _G_IB
            roff = pl.multiple_of(off // _G_CH, 8)
            pltpu.sync_copy(adj_hbm.at[pl.ds(roff, nsub)], idxa)
            g = {0: pltpu.async_copy(cv_hbm.at[idxa.at[0]], bufs[0], gsem[0])}
            for j in range(nsub):
                b = j & 1
                nb_ = (j + 1) & 1
                if j + 1 < nsub:
                    g[j + 1] = pltpu.async_copy(cv_hbm.at[idxa.at[j + 1]],
                                                bufs[nb_], gsem[nb_])
                g[j].wait()
                pltpu.sync_copy(bufs[b],
                                cvg_hbm.at[pl.ds(off + j * _G_CH, _G_CH)])
            return 0

        lax.fori_loop(0, n_blocks, block, 0)

    f = pl.kernel(
        body,
        out_type=jax.ShapeDtypeStruct((EP, H), jnp.int32),
        scratch_types=[
            pltpu.VMEM((nsub, _G_CH), jnp.int32),
            pltpu.VMEM((_G_CH, H), jnp.int32),
            pltpu.VMEM((_G_CH, H), jnp.int32),
            pltpu.SemaphoreType.DMA,
            pltpu.SemaphoreType.DMA,
            pltpu.SemaphoreType.DMA,
            pltpu.SemaphoreType.DMA,
        ],
        **_SC_MESH,
    )
    return f(cv, adj_flat_p.reshape(EP // _G_CH, _G_CH))


def _sc_gather_bond(bond, abaj_flat_p):
    rows_w = EP // NW
    n_blocks = rows_w // _G_IB
    nsub = _G_IB // _G_CH

    def body(bond_hbm, abaj_hbm, bondg_hbm, idxb, bb0, bb1, sg0, sg1):
        base = _wid() * rows_w
        bufs = (bb0, bb1)
        gsem = (sg0, sg1)

        def block(bi, _):
            off = base + bi * _G_IB
            roff = pl.multiple_of(off // _G_CH, 8)
            pltpu.sync_copy(abaj_hbm.at[pl.ds(roff, nsub)], idxb)
            g = {0: pltpu.async_copy(bond_hbm.at[idxb.at[0]], bufs[0],
                                     gsem[0])}
            for j in range(nsub):
                b = j & 1
                nb_ = (j + 1) & 1
                if j + 1 < nsub:
                    g[j + 1] = pltpu.async_copy(bond_hbm.at[idxb.at[j + 1]],
                                                bufs[nb_], gsem[nb_])
                g[j].wait()
                pltpu.sync_copy(bufs[b],
                                bondg_hbm.at[pl.ds(off + j * _G_CH, _G_CH)])
            return 0

        lax.fori_loop(0, n_blocks, block, 0)

    f = pl.kernel(
        body,
        out_type=jax.ShapeDtypeStruct((EP, DE), jnp.float32),
        scratch_types=[
            pltpu.VMEM((nsub, _G_CH), jnp.int32),
            pltpu.VMEM((_G_CH, DE), jnp.float32),
            pltpu.VMEM((_G_CH, DE), jnp.float32),
            pltpu.SemaphoreType.DMA,
            pltpu.SemaphoreType.DMA,
        ],
        compiler_params=pltpu.CompilerParams(use_tc_tiling_on_sc=False),
        **_SC_MESH,
    )
    return f(bond, abaj_flat_p.reshape(EP // _G_CH, _G_CH))


def _transforms_body(atom_ref, w_ref, b_ref, cv_ref, bu_ref):
    x = atom_ref[...]
    y = jnp.dot(x, w_ref[...], preferred_element_type=jnp.float32) + b_ref[...]
    cbits = lax.bitcast_convert_type(y[:, :H], jnp.int32)
    vbits = lax.bitcast_convert_type(y[:, H:256], jnp.int32)
    cv_ref[...] = ((cbits + 0x8000) & jnp.int32(-65536)) | \
        lax.shift_right_logical(vbits + 0x8000, 16)
    bu_ref[...] = y[:, 256:]


def _transforms(atom_pad, w_all, b_all):
    BA = 512
    grid = (N_PAD // BA,)
    return pl.pallas_call(
        _transforms_body,
        grid=grid,
        in_specs=[
            pl.BlockSpec((BA, D), lambda i: (i, 0)),
            pl.BlockSpec((D, 512), lambda i: (0, 0)),
            pl.BlockSpec((1, 512), lambda i: (0, 0)),
        ],
        out_specs=[
            pl.BlockSpec((BA, H), lambda i: (i, 0)),
            pl.BlockSpec((BA, 256), lambda i: (i, 0)),
        ],
        out_shape=[
            jax.ShapeDtypeStruct((N_PAD, H), jnp.int32),
            jax.ShapeDtypeStruct((N_PAD, 256), jnp.float32),
        ],
    )(atom_pad, w_all, b_all)


_W_BLK = 8
_W_G = 8
_W_NR = NW // _W_G
_W_RANGE = EP // _W_NR
_W_SLC = _W_RANGE // _W_G


def _sc_winner(abaj_w):
    rows_m = (EP // 128) // _W_G
    n_blocks = rows_m // _W_BLK

    def body(abaj_hbm, stage_hbm, win, idxw, sem):
        c = lax.axis_index("c")
        s = lax.axis_index("s")
        g = s // _W_G
        m = s % _W_G
        lo = (c * 2 + g) * _W_RANGE
        iota = lax.iota(jnp.int32, 16)

        def init(i, _):
            for u in range(8):
                win[pl.ds((i * 8 + u) * 16, 16)] = jnp.full((16,), -1,
                                                            jnp.int32)
            return 0

        lax.fori_loop(0, _W_RANGE // 128, init, 0)

        row0 = m * rows_m

        def block(bi, _):
            roff = pl.multiple_of(row0 + bi * _W_BLK, 8)
            pltpu.sync_copy(abaj_hbm.at[pl.ds(roff, _W_BLK)], idxw)

            def row(r, _):
                for cc in range(8):
                    k_base = (row0 + bi * _W_BLK + r) * 128 + cc * 16
                    e_vec = idxw[r, pl.ds(cc * 16, 16)]
                    k_vec = k_base + iota
                    inr = (e_vec >= lo) & (e_vec < lo + _W_RANGE)
                    rel = jnp.where(inr, e_vec - lo, 0)
                    plsc.store_scatter(win, [rel], k_vec, mask=inr)
                    stored = plsc.load_gather(win, [rel], mask=inr)
                    needi = (inr & (stored < k_vec)).astype(jnp.int32)

                    def fix_cond(cr):
                        return cr[0] > 0

                    def fix_body(cr):
                        _, ni = cr
                        mm = ni > 0
                        plsc.store_scatter(win, [rel], k_vec, mask=mm)
                        st = plsc.load_gather(win, [rel], mask=mm)
                        ni2 = (mm & (st < k_vec)).astype(jnp.int32)
                        return (jnp.sum(ni2), ni2)

                    lax.while_loop(fix_cond, fix_body,
                                   (jnp.sum(needi), needi))
                return 0

            lax.fori_loop(0, _W_BLK, row, 0)
            return 0

        lax.fori_loop(0, n_blocks, block, 0)

        pltpu.sync_copy(win, stage_hbm.at[pl.ds((c * NS + s) * _W_RANGE, _W_RANGE)])

    f = pl.kernel(
        body,
        out_type=jax.ShapeDtypeStruct((NW * _W_RANGE,), jnp.int32),
        scratch_types=[
            pltpu.VMEM((_W_RANGE,), jnp.int32),
            pltpu.VMEM((_W_BLK, 128), jnp.int32),
            pltpu.SemaphoreType.DMA,
        ],
        compiler_params=pltpu.CompilerParams(needs_layout_passes=False),
        **_SC_MESH,
    )
    stage = f(abaj_w)

    def mbody(stage_hbm, win_hbm, tmp, outb, sem):
        c = lax.axis_index("c")
        s = lax.axis_index("s")
        g = s // _W_G
        m = s % _W_G
        lo = (c * 2 + g) * _W_RANGE
        sl = m * _W_SLC
        out_lo = lo + sl
        iota = lax.iota(jnp.int32, 16)
        pltpu.sync_copy(
            stage_hbm.at[pl.ds((c * NS + g * _W_G) * _W_RANGE + sl, _W_SLC)],
            outb)
        for src in range(1, _W_G):
            pltpu.sync_copy(
                stage_hbm.at[pl.ds((c * NS + g * _W_G + src) * _W_RANGE + sl,
                                   _W_SLC)], tmp)

            def mx(i, _):
                for u in range(8):
                    o = (i * 8 + u) * 16
                    outb[pl.ds(o, 16)] = jnp.maximum(outb[pl.ds(o, 16)],
                                                     tmp[pl.ds(o, 16)])
                return 0

            lax.fori_loop(0, _W_SLC // 128, mx, 0)

        def fin(i, _):
            for u in range(8):
                o = (i * 8 + u) * 16
                v = outb[pl.ds(o, 16)]
                sent = E + lax.rem(out_lo + o + iota, NZROWS)
                outb[pl.ds(o, 16)] = jnp.where(v < 0, sent, v)
            return 0

        lax.fori_loop(0, _W_SLC // 128, fin, 0)
        pltpu.sync_copy(outb, win_hbm.at[pl.ds(out_lo, _W_SLC)])

    fm = pl.kernel(
        mbody,
        out_type=jax.ShapeDtypeStruct((EP,), jnp.int32),
        scratch_types=[
            pltpu.VMEM((_W_SLC,), jnp.int32),
            pltpu.VMEM((_W_SLC,), jnp.int32),
            pltpu.SemaphoreType.DMA,
        ],
        **_SC_MESH,
    )
    return fm(stage)


def _sc_bond_gather(winner, nb_flat):
    n_groups = E // 128

    def body(win_hbm, nb_hbm, out_hbm, idx0, idx1, db0, db1, sg0, sg1):
        wid = _wid()
        per = n_groups // NW
        extra = n_groups - per * NW
        base_g = wid * per
        idxs = (idx0, idx1)
        bufs = (db0, db1)
        gsem = (sg0, sg1)

        def pair(pi, _):
            off0 = pl.multiple_of((base_g + 2 * pi) * 128, 128)
            off1 = pl.multiple_of((base_g + 2 * pi + 1) * 128, 128)
            pltpu.sync_copy(win_hbm.at[pl.ds(off0, 128)], idxs[0])
            g0 = pltpu.async_copy(nb_hbm.at[idxs[0]], bufs[0], gsem[0])
            pltpu.sync_copy(win_hbm.at[pl.ds(off1, 128)], idxs[1])
            g1 = pltpu.async_copy(nb_hbm.at[idxs[1]], bufs[1], gsem[1])
            g0.wait()
            pltpu.sync_copy(bufs[0], out_hbm.at[pl.ds(off0, 128)])
            g1.wait()
            pltpu.sync_copy(bufs[1], out_hbm.at[pl.ds(off1, 128)])
            return 0

        lax.fori_loop(0, per // 2, pair, 0)

        @pl.when(wid < extra)
        def _tail():
            off = pl.multiple_of((NW * per + wid) * 128, 128)
            pltpu.sync_copy(win_hbm.at[pl.ds(off, 128)], idxs[0])
            pltpu.async_copy(nb_hbm.at[idxs[0]], bufs[0], gsem[0]).wait()
            pltpu.sync_copy(bufs[0], out_hbm.at[pl.ds(off, 128)])

    f = pl.kernel(
        body,
        out_type=jax.ShapeDtypeStruct((E, H), jnp.float32),
        scratch_types=[
            pltpu.VMEM((128,), jnp.int32),
            pltpu.VMEM((128,), jnp.int32),
            pltpu.VMEM((128, H), jnp.float32),
            pltpu.VMEM((128, H), jnp.float32),
            pltpu.SemaphoreType.DMA,
            pltpu.SemaphoreType.DMA,
        ],
        **_SC_MESH,
    )
    return f(winner, nb_flat)


_BA = 512


def _edge_body(cvg_ref, bondg_ref, bu_ref, awt_ref, ab_ref, nb_ref, ao_ref):
    i = pl.program_id(0)
    px = cvg_ref[...]
    cx = lax.bitcast_convert_type(px & jnp.int32(-65536), jnp.float32)
    vn = lax.bitcast_convert_type(lax.shift_left(px, 16), jnp.float32)
    bg = bondg_ref[...].reshape(_BA * DEG, DE)
    ae = jnp.dot(bg, awt_ref[...], preferred_element_type=jnp.float32)
    syn = ae.reshape(_BA, DEG, H) + ab_ref[...][None, :, :] + cx \
        + bu_ref[:, :H][:, None, :]
    ids = i * _BA + lax.broadcasted_iota(jnp.int32, (_BA, 1, 1), 0)
    nb_ref[...] = jnp.where(ids < N, jax.nn.relu(syn), 0.0)
    gates = jax.nn.sigmoid(syn)
    agg = jnp.sum(gates * vn, axis=1)
    ao_ref[...] = jax.nn.relu(bu_ref[:, H:] + agg)


def _tc_edge(cvg, bondg, bu, A_w, A_b):
    grid = (N_PAD // _BA,)
    nb, ao = pl.pallas_call(
        _edge_body,
        grid=grid,
        in_specs=[
            pl.BlockSpec((_BA, DEG, H), lambda i: (i, 0, 0)),
            pl.BlockSpec((_BA, DEG, DE), lambda i: (i, 0, 0)),
            pl.BlockSpec((_BA, 256), lambda i: (i, 0)),
            pl.BlockSpec((DE, H), lambda i: (0, 0)),
            pl.BlockSpec((1, H), lambda i: (0, 0)),
        ],
        out_specs=[
            pl.BlockSpec((_BA, DEG, H), lambda i: (i, 0, 0)),
            pl.BlockSpec((_BA, H), lambda i: (i, 0)),
        ],
        out_shape=[
            jax.ShapeDtypeStruct((N_PAD, DEG, H), jnp.float32),
            jax.ShapeDtypeStruct((N_PAD, H), jnp.float32),
        ],
    )(cvg.reshape(N_PAD, DEG, H), bondg.reshape(N_PAD, DEG, DE), bu,
      A_w.T, A_b[None, :])
    return nb, ao


def kernel(atom_feature_matrix, bond_feature_matrix, atom_adjacency_list,
           atom_bond_adjacency_list, U_w, U_b, V_w, V_b, A_w, A_b, B_w, B_b,
           C_w, C_b):
    atom_pad = jnp.concatenate(
        [atom_feature_matrix, jnp.zeros((N_PAD - N, D), jnp.float32)], axis=0)
    w_all = jnp.concatenate([C_w.T, V_w.T, B_w.T, U_w.T], axis=1)
    b_all = jnp.concatenate([C_b, V_b, B_b, U_b])[None, :]

    cv, bu = _transforms(atom_pad, w_all, b_all)

    adj_flat = atom_adjacency_list.reshape(-1)
    abaj_flat = atom_bond_adjacency_list.reshape(-1)

    pad_ids = (jnp.arange(N_PAD * DEG - E, dtype=jnp.int32) % N)
    adj_flat_p = jnp.concatenate([adj_flat, pad_ids])
    abaj_flat_p = jnp.concatenate([abaj_flat, pad_ids])

    cvg = _sc_gather(cv, bond_feature_matrix, adj_flat_p, abaj_flat_p)
    bondg = _sc_gather_bond(bond_feature_matrix, abaj_flat_p)

    nb, ao = _tc_edge(cvg, bondg, bu, A_w, A_b)
    atom_out = ao[:N]

    abaj_w = jnp.concatenate(
        [abaj_flat, jnp.full((EP - E,), E, jnp.int32)]).reshape(EP // 128, 128)
    winner = _sc_winner(abaj_w)
    bond_out = _sc_bond_gather(winner, nb.reshape(EP, H))
    return (atom_out, bond_out)
